# baseline (speedup 1.0000x reference)
import math

import jax
import jax.numpy as jnp
from jax import lax
from jax.experimental import pallas as pl
from jax.experimental.pallas import tpu as pltpu
from jax.experimental.pallas import tpu_sc as plsc

D_MODEL = 768
SCALE = math.sqrt(D_MODEL)

NUM_CORES = 2
NUM_SUBCORES = 16
NUM_WORKERS = NUM_CORES * NUM_SUBCORES
LANES = 16

TOTAL_IDS = 4 * 2048
IDS_PER_WORKER = TOTAL_IDS // NUM_WORKERS
CHUNK = 32
NUM_CHUNKS = IDS_PER_WORKER // CHUNK
NBUF = 5
SLICES_PER_ROW = D_MODEL // LANES


def _body(table_hbm, ids_hbm, out_hbm, idx_v, rows, gsem, osem):
    wid = lax.axis_index("s") * NUM_CORES + lax.axis_index("c")
    base = wid * IDS_PER_WORKER

    pltpu.sync_copy(ids_hbm.at[pl.ds(base, IDS_PER_WORKER)], idx_v)

    def gather(ci):
        b = ci % NBUF
        return pltpu.make_async_copy(
            table_hbm.at[idx_v.at[pl.ds(ci * CHUNK, CHUNK)]], rows[b], gsem[b]
        )

    def writeback(ci):
        b = ci % NBUF
        return pltpu.make_async_copy(
            rows[b], out_hbm.at[pl.ds(base + ci * CHUNK, CHUNK)], osem[b]
        )

    for ci in range(NBUF - 1):
        gather(ci).start()

    for ci in range(NUM_CHUNKS):
        b = ci % NBUF
        gather(ci).wait()

        rows_b = rows[b]

        def scale_rows(i, _, rows_b=rows_b):
            r = i * 2
            for rr in range(2):
                for j in range(SLICES_PER_ROW):
                    sl = pl.ds(j * LANES, LANES)
                    rows_b[r + rr, sl] = rows_b[r + rr, sl] * SCALE
            return _

        lax.fori_loop(0, CHUNK // 2, scale_rows, 0)
        writeback(ci).start()

        nxt = ci + NBUF - 1
        if nxt < NUM_CHUNKS:
            if nxt >= NBUF:
                writeback(nxt - NBUF).wait()
            gather(nxt).start()

    for ci in range(max(0, NUM_CHUNKS - NBUF), NUM_CHUNKS):
        writeback(ci).wait()


@jax.jit
def _embed(table, ids):
    mesh = plsc.VectorSubcoreMesh(core_axis_name="c", subcore_axis_name="s")
    return pl.kernel(
        _body,
        out_type=jax.ShapeDtypeStruct((TOTAL_IDS, D_MODEL), jnp.float32),
        mesh=mesh,
        scratch_types=[
            pltpu.VMEM((IDS_PER_WORKER,), jnp.int32),
            [pltpu.VMEM((CHUNK, D_MODEL), jnp.float32) for _ in range(NBUF)],
            [pltpu.SemaphoreType.DMA for _ in range(NBUF)],
            [pltpu.SemaphoreType.DMA for _ in range(NBUF)],
        ],
    )(table, ids)


def kernel(token_ids, embedding_table):
    ids = token_ids.reshape(-1).astype(jnp.int32)
    out = _embed(embedding_table, ids)
    return out.reshape(token_ids.shape + (D_MODEL,))


# speedup vs baseline: 1.1188x; 1.1188x over previous
import math

import jax
import jax.numpy as jnp
from jax import lax
from jax.experimental import pallas as pl
from jax.experimental.pallas import tpu as pltpu
from jax.experimental.pallas import tpu_sc as plsc

D_MODEL = 768
SCALE = math.sqrt(D_MODEL)

NUM_CORES = 2
NUM_SUBCORES = 16
NUM_WORKERS = NUM_CORES * NUM_SUBCORES
LANES = 16

TOTAL_IDS = 4 * 2048
IDS_PER_WORKER = TOTAL_IDS // NUM_WORKERS
CHUNK = 32
NUM_CHUNKS = IDS_PER_WORKER // CHUNK
NBUF = 5
SLICES_PER_ROW = D_MODEL // LANES


def _body(table_hbm, ids_hbm, out_hbm, idx_v, rows3, gsem, osem):
    wid = lax.axis_index("s") * NUM_CORES + lax.axis_index("c")
    base = wid * IDS_PER_WORKER

    pltpu.sync_copy(ids_hbm.at[pl.ds(base, IDS_PER_WORKER)], idx_v)

    def gather(ci, b):
        return pltpu.make_async_copy(
            table_hbm.at[idx_v.at[pl.ds(ci * CHUNK, CHUNK)]],
            rows3.at[b],
            gsem.at[b],
        )

    def writeback(ci, b):
        return pltpu.make_async_copy(
            rows3.at[b],
            out_hbm.at[pl.ds(base + ci * CHUNK, CHUNK)],
            osem.at[b],
        )

    def prime(ci, _):
        gather(ci, lax.rem(ci, NBUF)).start()
        return _

    lax.fori_loop(0, NBUF - 1, prime, 0)

    def chunk_step(ci, _):
        b = lax.rem(ci, NBUF)
        gather(ci, b).wait()

        def scale_row(r, _2):
            for j in range(SLICES_PER_ROW):
                sl = pl.ds(j * LANES, LANES)
                rows3[b, r, sl] = rows3[b, r, sl] * SCALE
            return _2

        lax.fori_loop(0, CHUNK, scale_row, 0)
        writeback(ci, b).start()

        nxt = ci + NBUF - 1

        @pl.when(nxt < NUM_CHUNKS)
        def _issue():
            nb = lax.rem(nxt, NBUF)

            @pl.when(nxt >= NBUF)
            def _drain():
                writeback(nxt - NBUF, nb).wait()

            gather(nxt, nb).start()

        return _

    lax.fori_loop(0, NUM_CHUNKS, chunk_step, 0)

    def drain(ci, _):
        writeback(ci, lax.rem(ci, NBUF)).wait()
        return _

    lax.fori_loop(NUM_CHUNKS - NBUF, NUM_CHUNKS, drain, 0)


@jax.jit
def _embed(table, ids):
    mesh = plsc.VectorSubcoreMesh(core_axis_name="c", subcore_axis_name="s")
    return pl.kernel(
        _body,
        out_type=jax.ShapeDtypeStruct((TOTAL_IDS, D_MODEL), jnp.float32),
        mesh=mesh,
        scratch_types=[
            pltpu.VMEM((IDS_PER_WORKER,), jnp.int32),
            pltpu.VMEM((NBUF, CHUNK, D_MODEL), jnp.float32),
            pltpu.SemaphoreType.DMA((NBUF,)),
            pltpu.SemaphoreType.DMA((NBUF,)),
        ],
    )(table, ids)


def kernel(token_ids, embedding_table):
    ids = token_ids.reshape(-1).astype(jnp.int32)
    out = _embed(embedding_table, ids)
    return out.reshape(token_ids.shape + (D_MODEL,))


# speedup vs baseline: 1.1210x; 1.0020x over previous
import math

import jax
import jax.numpy as jnp
from jax import lax
from jax.experimental import pallas as pl
from jax.experimental.pallas import tpu as pltpu
from jax.experimental.pallas import tpu_sc as plsc

D_MODEL = 768
SCALE = math.sqrt(D_MODEL)

NUM_CORES = 2
NUM_SUBCORES = 16
NUM_WORKERS = NUM_CORES * NUM_SUBCORES
LANES = 16

TOTAL_IDS = 4 * 2048
IDS_PER_WORKER = TOTAL_IDS // NUM_WORKERS
CHUNK = 32
NUM_CHUNKS = IDS_PER_WORKER // CHUNK
NBUF = 5
SLICES_PER_ROW = D_MODEL // LANES


def _body(table_hbm, ids_hbm, out_hbm, idx_v, rows3, gsem, osem):
    wid = lax.axis_index("s") * NUM_CORES + lax.axis_index("c")
    base = wid * IDS_PER_WORKER

    pltpu.sync_copy(ids_hbm.at[pl.ds(base, IDS_PER_WORKER)], idx_v)

    def gather(ci, b):
        return pltpu.make_async_copy(
            table_hbm.at[idx_v.at[pl.ds(ci * CHUNK, CHUNK)]],
            rows3.at[b],
            gsem.at[b],
        )

    def writeback(ci, b):
        return pltpu.make_async_copy(
            rows3.at[b],
            out_hbm.at[pl.ds(base + ci * CHUNK, CHUNK)],
            osem.at[b],
        )

    def prime(ci, _):
        gather(ci, lax.rem(ci, NBUF)).start()
        return _

    lax.fori_loop(0, NBUF - 1, prime, 0)

    def chunk_step(ci, _):
        b = lax.rem(ci, NBUF)
        gather(ci, b).wait()

        def scale_row(r, _2):
            def scale_grp(g, _3):
                for jj in range(8):
                    sl = pl.ds(g * (8 * LANES) + jj * LANES, LANES)
                    rows3[b, r, sl] = rows3[b, r, sl] * SCALE
                return _3

            lax.fori_loop(0, SLICES_PER_ROW // 8, scale_grp, 0)
            return _2

        lax.fori_loop(0, CHUNK, scale_row, 0)
        writeback(ci, b).start()

        nxt = ci + NBUF - 1

        @pl.when(nxt < NUM_CHUNKS)
        def _issue():
            nb = lax.rem(nxt, NBUF)

            @pl.when(nxt >= NBUF)
            def _drain():
                writeback(nxt - NBUF, nb).wait()

            gather(nxt, nb).start()

        return _

    lax.fori_loop(0, NUM_CHUNKS, chunk_step, 0)

    def drain(ci, _):
        writeback(ci, lax.rem(ci, NBUF)).wait()
        return _

    lax.fori_loop(NUM_CHUNKS - NBUF, NUM_CHUNKS, drain, 0)


@jax.jit
def _embed(table, ids):
    mesh = plsc.VectorSubcoreMesh(core_axis_name="c", subcore_axis_name="s")
    return pl.kernel(
        _body,
        out_type=jax.ShapeDtypeStruct((TOTAL_IDS, D_MODEL), jnp.float32),
        mesh=mesh,
        scratch_types=[
            pltpu.VMEM((IDS_PER_WORKER,), jnp.int32),
            pltpu.VMEM((NBUF, CHUNK, D_MODEL), jnp.float32),
            pltpu.SemaphoreType.DMA((NBUF,)),
            pltpu.SemaphoreType.DMA((NBUF,)),
        ],
    )(table, ids)


def kernel(token_ids, embedding_table):
    ids = token_ids.reshape(-1).astype(jnp.int32)
    out = _embed(embedding_table, ids)
    return out.reshape(token_ids.shape + (D_MODEL,))


# speedup vs baseline: 1.1387x; 1.0158x over previous
import math

import jax
import jax.numpy as jnp
from jax import lax
from jax.experimental import pallas as pl
from jax.experimental.pallas import tpu as pltpu
from jax.experimental.pallas import tpu_sc as plsc

D_MODEL = 768
SCALE = math.sqrt(D_MODEL)

NUM_CORES = 2
NUM_SUBCORES = 16
NUM_WORKERS = NUM_CORES * NUM_SUBCORES
LANES = 16

TOTAL_IDS = 4 * 2048
IDS_PER_WORKER = TOTAL_IDS // NUM_WORKERS
CHUNK = 16
NUM_CHUNKS = IDS_PER_WORKER // CHUNK
NBUF = 10
SLICES_PER_ROW = D_MODEL // LANES


def _body(table_hbm, ids_hbm, out_hbm, idx_v, rows3, gsem, osem):
    wid = lax.axis_index("s") * NUM_CORES + lax.axis_index("c")
    base = wid * IDS_PER_WORKER

    pltpu.sync_copy(ids_hbm.at[pl.ds(base, IDS_PER_WORKER)], idx_v)

    def gather(ci, b):
        return pltpu.make_async_copy(
            table_hbm.at[idx_v.at[pl.ds(ci * CHUNK, CHUNK)]],
            rows3.at[b],
            gsem.at[b],
        )

    def writeback(ci, b):
        return pltpu.make_async_copy(
            rows3.at[b],
            out_hbm.at[pl.ds(base + ci * CHUNK, CHUNK)],
            osem.at[b],
        )

    def prime(ci, _):
        gather(ci, lax.rem(ci, NBUF)).start()
        return _

    lax.fori_loop(0, NBUF - 1, prime, 0)

    def chunk_step(ci, _):
        b = lax.rem(ci, NBUF)
        gather(ci, b).wait()

        def scale_row(r, _2):
            def scale_grp(g, _3):
                for jj in range(8):
                    sl = pl.ds(g * (8 * LANES) + jj * LANES, LANES)
                    rows3[b, r, sl] = rows3[b, r, sl] * SCALE
                return _3

            lax.fori_loop(0, SLICES_PER_ROW // 8, scale_grp, 0)
            return _2

        lax.fori_loop(0, CHUNK, scale_row, 0)
        writeback(ci, b).start()

        nxt = ci + NBUF - 1

        @pl.when(nxt < NUM_CHUNKS)
        def _issue():
            nb = lax.rem(nxt, NBUF)

            @pl.when(nxt >= NBUF)
            def _drain():
                writeback(nxt - NBUF, nb).wait()

            gather(nxt, nb).start()

        return _

    lax.fori_loop(0, NUM_CHUNKS, chunk_step, 0)

    def drain(ci, _):
        writeback(ci, lax.rem(ci, NBUF)).wait()
        return _

    lax.fori_loop(NUM_CHUNKS - NBUF, NUM_CHUNKS, drain, 0)


@jax.jit
def _embed(table, ids):
    mesh = plsc.VectorSubcoreMesh(core_axis_name="c", subcore_axis_name="s")
    return pl.kernel(
        _body,
        out_type=jax.ShapeDtypeStruct((TOTAL_IDS, D_MODEL), jnp.float32),
        mesh=mesh,
        scratch_types=[
            pltpu.VMEM((IDS_PER_WORKER,), jnp.int32),
            pltpu.VMEM((NBUF, CHUNK, D_MODEL), jnp.float32),
            pltpu.SemaphoreType.DMA((NBUF,)),
            pltpu.SemaphoreType.DMA((NBUF,)),
        ],
    )(table, ids)


def kernel(token_ids, embedding_table):
    ids = token_ids.reshape(-1).astype(jnp.int32)
    out = _embed(embedding_table, ids)
    return out.reshape(token_ids.shape + (D_MODEL,))


# speedup vs baseline: 1.1585x; 1.0173x over previous
import math

import jax
import jax.numpy as jnp
from jax import lax
from jax.experimental import pallas as pl
from jax.experimental.pallas import tpu as pltpu
from jax.experimental.pallas import tpu_sc as plsc

D_MODEL = 768
SCALE = math.sqrt(D_MODEL)

NUM_CORES = 2
NUM_SUBCORES = 16
NUM_WORKERS = NUM_CORES * NUM_SUBCORES
LANES = 16

TOTAL_IDS = 4 * 2048
IDS_PER_WORKER = TOTAL_IDS // NUM_WORKERS
CHUNK = 16
NUM_CHUNKS = IDS_PER_WORKER // CHUNK
NBUF = 8
SLICES_PER_ROW = D_MODEL // LANES


def _body(table_hbm, ids_hbm, out_hbm, idx_v, rows3, gsem, osem):
    wid = lax.axis_index("s") * NUM_CORES + lax.axis_index("c")
    base = wid * IDS_PER_WORKER

    pltpu.sync_copy(ids_hbm.at[pl.ds(base, IDS_PER_WORKER)], idx_v)

    def gather(ci, b):
        return pltpu.make_async_copy(
            table_hbm.at[idx_v.at[pl.ds(ci * CHUNK, CHUNK)]],
            rows3.at[b],
            gsem.at[b],
        )

    def writeback(ci, b):
        return pltpu.make_async_copy(
            rows3.at[b],
            out_hbm.at[pl.ds(base + ci * CHUNK, CHUNK)],
            osem.at[b],
        )

    def prime(ci, _):
        gather(ci, lax.rem(ci, NBUF)).start()
        return _

    lax.fori_loop(0, NBUF - 1, prime, 0)

    def chunk_step(ci, _):
        b = lax.rem(ci, NBUF)
        gather(ci, b).wait()

        def scale_row(r, _2):
            def scale_grp(g, _3):
                for jj in range(8):
                    sl = pl.ds(g * (8 * LANES) + jj * LANES, LANES)
                    rows3[b, r, sl] = rows3[b, r, sl] * SCALE
                return _3

            lax.fori_loop(0, SLICES_PER_ROW // 8, scale_grp, 0)
            return _2

        lax.fori_loop(0, CHUNK, scale_row, 0)
        writeback(ci, b).start()

        nxt = ci + NBUF - 1

        @pl.when(nxt < NUM_CHUNKS)
        def _issue():
            nb = lax.rem(nxt, NBUF)

            @pl.when(nxt >= NBUF)
            def _drain():
                writeback(nxt - NBUF, nb).wait()

            gather(nxt, nb).start()

        return _

    lax.fori_loop(0, NUM_CHUNKS, chunk_step, 0)

    def drain(ci, _):
        writeback(ci, lax.rem(ci, NBUF)).wait()
        return _

    lax.fori_loop(NUM_CHUNKS - NBUF, NUM_CHUNKS, drain, 0)


@jax.jit
def _embed(table, ids):
    mesh = plsc.VectorSubcoreMesh(core_axis_name="c", subcore_axis_name="s")
    return pl.kernel(
        _body,
        out_type=jax.ShapeDtypeStruct((TOTAL_IDS, D_MODEL), jnp.float32),
        mesh=mesh,
        scratch_types=[
            pltpu.VMEM((IDS_PER_WORKER,), jnp.int32),
            pltpu.VMEM((NBUF, CHUNK, D_MODEL), jnp.float32),
            pltpu.SemaphoreType.DMA((NBUF,)),
            pltpu.SemaphoreType.DMA((NBUF,)),
        ],
    )(table, ids)


def kernel(token_ids, embedding_table):
    ids = token_ids.reshape(-1).astype(jnp.int32)
    out = _embed(embedding_table, ids)
    return out.reshape(token_ids.shape + (D_MODEL,))
